# baseline (speedup 1.0000x reference)
import jax
import jax.numpy as jnp
from jax import lax
from jax.experimental import pallas as pl
from jax.experimental.pallas import tpu as pltpu
from jax.experimental.pallas import tpu_sc as plsc

NA = 10000
NP = 160000
C = 128
PIW = 64
NB = 8
NPAD = 163840
NCHUNK = NPAD // 128
NA_ACC = 10112
RPT = NA_ACC // 16
CPW = NCHUNK // 32
CPW64 = (NPAD // 64) // 32
NCH64 = NPAD // 64
CH_C0 = 640
TPC0 = CH_C0 // 16
TPC1 = (NCH64 - CH_C0) // 16
PAIR_BLK = 2048
ATOM_BLK = 1000
HC = C // 2

_SC_MESH = dict(core_axis_name="c", subcore_axis_name="s")


def _rn16(v):
    return (lax.bitcast_convert_type(v, jnp.int32) + 0x8000) >> 16


def _pack2(lo, hi):
    return (_rn16(hi) << 16) | (_rn16(lo) & 0xFFFF)


def _unpack2(p):
    lo = lax.bitcast_convert_type(p << 16, jnp.float32)
    hi = lax.bitcast_convert_type(p & jnp.int32(-65536), jnp.float32)
    return lo, hi


def _drain(dummy_hbm, dst_ref, sem):
    pltpu.make_async_copy(dummy_hbm, dst_ref, sem).wait()


def _gi_body(h_hbm, ii_hbm, oi_hbm, ii_v, bi, sg, sw):
    c = lax.axis_index("c")
    s = lax.axis_index("s")
    ch0 = jnp.where(c == 1, s * TPC0, CH_C0 + s * TPC1)
    cnt = jnp.where(c == 1, TPC0, TPC1)
    pltpu.sync_copy(ii_hbm.at[pl.ds(ch0, TPC1)], ii_v)
    dummy = h_hbm.at[pl.ds(0, 64)]

    @pl.loop(0, cnt, step=8)
    def _(k):
        for b in range(8):
            pltpu.async_copy(h_hbm.at[ii_v.at[k + b]], bi.at[b], sg)
        for b in range(8):
            _drain(dummy, bi.at[b], sg)
        for b in range(8):
            row = (ch0 + k + b) * 64
            pltpu.async_copy(bi.at[b], oi_hbm.at[pl.ds(row, 64)], sw)
        for b in range(8):
            _drain(dummy, bi.at[b], sw)


def _gather_hi(h, ii_g64):
    return pl.kernel(
        _gi_body,
        out_type=jax.ShapeDtypeStruct((NPAD, C), jnp.float32),
        mesh=plsc.VectorSubcoreMesh(**_SC_MESH),
        scratch_types=[
            pltpu.VMEM((TPC1, 64), jnp.int32),
            pltpu.VMEM((8, 64, C), jnp.float32),
            pltpu.SemaphoreType.DMA,
            pltpu.SemaphoreType.DMA,
        ],
        name="sc_gather_hi",
    )(h, ii_g64)


def _gj_body(t_hbm, jj_hbm, out_hbm, jj_v, buf, sg, sw):
    c = lax.axis_index("c")
    s = lax.axis_index("s")
    ch0 = jnp.where(c == 1, s * TPC0, CH_C0 + s * TPC1)
    cnt = jnp.where(c == 1, TPC0, TPC1)
    pltpu.sync_copy(jj_hbm.at[pl.ds(ch0, TPC1)], jj_v)
    dummy = t_hbm.at[pl.ds(0, 64)]

    @pl.loop(0, cnt, step=4)
    def _(k):
        for b in range(4):
            pltpu.async_copy(t_hbm.at[jj_v.at[k + b]], buf.at[b], sg)
        for b in range(4):
            _drain(dummy, buf.at[b], sg)
        for b in range(4):
            row = (ch0 + k + b) * 64
            pltpu.async_copy(buf.at[b], out_hbm.at[pl.ds(row, 64)], sw)
        for b in range(4):
            _drain(dummy, buf.at[b], sw)


def _gather_j(tbl, jj_g64):
    return pl.kernel(
        _gj_body,
        out_type=jax.ShapeDtypeStruct((NPAD, 4 * HC), jnp.int32),
        mesh=plsc.VectorSubcoreMesh(**_SC_MESH),
        scratch_types=[
            pltpu.VMEM((TPC1, 64), jnp.int32),
            pltpu.VMEM((4, 64, 4 * HC), jnp.int32),
            pltpu.SemaphoreType.DMA,
            pltpu.SemaphoreType.DMA,
        ],
        name="sc_gather_j",
    )(tbl, jj_g64)


def _scatter_rounds(vxs, ii_hbm, zeros, out, idx_v, vb, acc, sl, ss):
    c = lax.axis_index("c")
    s = lax.axis_index("s")
    r0 = s * RPT
    ch0 = (c * 16 + s) * CPW
    pltpu.sync_copy(ii_hbm.at[pl.ds(ch0, CPW)], idx_v)
    for x, vx in enumerate(vxs):
        pltpu.sync_copy(zeros, acc.at[pl.ds(r0, RPT)])
        plsc.subcore_barrier()
        dummy = vx.at[pl.ds(0, 128)]

        @pl.loop(0, CPW, step=2)
        def _(k, vx=vx, dummy=dummy):
            for b in range(2):
                pltpu.async_copy(
                    vx.at[pl.ds((ch0 + k + b) * 128, 128)], vb.at[b], sl)
            for b in range(2):
                _drain(dummy, vb.at[b], sl)
            for b in range(2):
                pltpu.async_copy(vb.at[b], acc.at[idx_v.at[k + b]], ss,
                                 add=True)
            for b in range(2):
                _drain(dummy, vb.at[b], ss)

        plsc.subcore_barrier()
        pltpu.sync_copy(acc.at[pl.ds(r0, RPT)], out.at[x, c, pl.ds(r0, RPT)])


def _sall_body(i1, x0, x1, x2, ii_hbm, zeros, out, idx_v, vb, acc, sl, ss):
    _scatter_rounds((i1, x0, x1, x2), ii_hbm, zeros, out, idx_v, vb, acc,
                    sl, ss)


def _scatter_sc(body, n, name, vals, ii_s, zeros):
    return pl.kernel(
        body,
        out_type=jax.ShapeDtypeStruct((n, 2, NA_ACC, C), jnp.float32),
        mesh=plsc.VectorSubcoreMesh(**_SC_MESH),
        scratch_types=[
            pltpu.VMEM((CPW, 128), jnp.int32),
            pltpu.VMEM((2, 128, C), jnp.float32),
            pltpu.VMEM_SHARED((NA_ACC, C), jnp.float32),
            pltpu.SemaphoreType.DMA,
            pltpu.SemaphoreType.DMA,
        ],
        name=name,
    )(*vals, ii_s, zeros)


def _t1_body(x, w0, b0, w1, b1, o, op):
    h = jnp.tanh(jnp.dot(x[...], w0[...], preferred_element_type=jnp.float32)
                 + b0[...])
    h = jnp.tanh(jnp.dot(h, w1[...], preferred_element_type=jnp.float32)
                 + b1[...])
    o[...] = h
    op[...] = _pack2(h[:, :HC], h[:, HC:])


def _atom_mlp(p1, w0, b0, w1, b1):
    full = lambda shape: pl.BlockSpec(shape, lambda i: (0, 0))
    return pl.pallas_call(
        _t1_body,
        grid=(NA // ATOM_BLK,),
        in_specs=[pl.BlockSpec((ATOM_BLK, C), lambda i: (i, 0)),
                  full((C, C)), full((1, C)), full((C, C)), full((1, C))],
        out_specs=[pl.BlockSpec((ATOM_BLK, C), lambda i: (i, 0)),
                   pl.BlockSpec((ATOM_BLK, HC), lambda i: (i, 0))],
        out_shape=[jax.ShapeDtypeStruct((NA, C), jnp.float32),
                   jax.ShapeDtypeStruct((NA, HC), jnp.int32)],
    )(p1, w0, b0.reshape(1, C), w1, b1.reshape(1, C))


def _t2_body(hi, jblk, basis, d3p, wa, wb, pib, iiw, o, o0, o1, o2):
    jv = jblk[...]
    hjl, hjh = _unpack2(jv[:, 3 * HC:])
    hj = jnp.concatenate([hjl, hjh], axis=1).astype(jnp.bfloat16)
    hib = hi[...].astype(jnp.bfloat16)
    pre = (jnp.dot(hib, wa[...], preferred_element_type=jnp.float32)
           + jnp.dot(hj, wb[...], preferred_element_type=jnp.float32))
    t = jnp.tanh(pre + pib[...])
    r8 = lax.broadcasted_iota(jnp.int32, (NB, PIW * NB), 0)
    c8 = lax.broadcasted_iota(jnp.int32, (NB, PIW * NB), 1)
    em = (c8 % NB == r8).astype(jnp.float32)
    rw = lax.broadcasted_iota(jnp.int32, (PIW * NB, PIW), 1)
    cw = lax.broadcasted_iota(jnp.int32, (PIW * NB, PIW), 0)
    fm = (cw // NB == rw).astype(jnp.bfloat16)
    y = (t * jnp.dot(basis[...], em, preferred_element_type=jnp.float32)
         ).astype(jnp.bfloat16)
    ip = jnp.dot(y, fm, preferred_element_type=jnp.float32)
    i1v = jnp.tanh(jnp.dot(ip, iiw[...], preferred_element_type=jnp.float32))
    o[...] = i1v
    d3v = d3p[...]
    for x, ox in enumerate((o0, o1, o2)):
        lo, hi_ = _unpack2(jv[:, x * HC:(x + 1) * HC])
        comp = jnp.concatenate([lo, hi_], axis=1)
        ox[...] = (comp + d3v[:, x:x + 1]) * i1v


def _pair_net(hi, jg, basis_p, d3_p, wa, wb, pib, iiw):
    full = lambda shape: pl.BlockSpec(shape, lambda i: (0, 0))
    row = lambda w: pl.BlockSpec((PAIR_BLK, w), lambda i: (i, 0))
    return pl.pallas_call(
        _t2_body,
        grid=(NPAD // PAIR_BLK,),
        in_specs=[row(C), row(4 * HC), row(NB), row(NB),
                  full((C, PIW * NB)), full((C, PIW * NB)),
                  full((1, PIW * NB)), full((PIW, C))],
        out_specs=[row(C), row(C), row(C), row(C)],
        out_shape=[jax.ShapeDtypeStruct((NPAD, C), jnp.float32)] * 4,
    )(hi, jg, basis_p, d3_p, wa.astype(jnp.bfloat16),
      wb.astype(jnp.bfloat16), pib.reshape(1, PIW * NB), iiw)


def _t3_body(p1, p3f, pa0, pa1, px00, px01, px10, px11, px20, px21,
             wp0, wp1, we, dw, o1, o3, o3p):
    p1a = pa0[0, 0] + pa1[0, 0]
    p1n = jnp.tanh(jnp.dot(p1a, wp0[...], preferred_element_type=jnp.float32))
    p1n = jnp.tanh(jnp.dot(p1n, wp1[...], preferred_element_type=jnp.float32))
    wev = we[...]
    p3n = [jnp.dot(a[0, 0] + b[0, 0], wev, preferred_element_type=jnp.float32)
           for a, b in ((px00, px01), (px10, px11), (px20, px21))]
    dotted = p3n[0] * p3n[0] + p3n[1] * p3n[1] + p3n[2] * p3n[2]
    p1o = p1n + jnp.dot(dotted, dw[...], preferred_element_type=jnp.float32)
    o1[...] = p1[...] + p1o
    p3fv = p3f[...]
    news = [p3fv[:, x * C:(x + 1) * C] + p3n[x] for x in range(3)]
    o3[...] = jnp.concatenate(news, axis=1)
    o3p[...] = jnp.concatenate(
        [_pack2(n[:, :HC], n[:, HC:]) for n in news], axis=1)


def _combine(p1, p3f, acc, wp0, wp1, we, dw):
    full = lambda shape: pl.BlockSpec(shape, lambda i: (0, 0))
    row = lambda w: pl.BlockSpec((ATOM_BLK, w), lambda i: (i, 0))
    px = lambda xx, cc: pl.BlockSpec(
        (1, 1, ATOM_BLK, C), lambda i, xx=xx, cc=cc: (xx, cc, i, 0))
    return pl.pallas_call(
        _t3_body,
        grid=(NA // ATOM_BLK,),
        in_specs=[row(C), row(3 * C), px(0, 0), px(0, 1),
                  px(1, 0), px(1, 1), px(2, 0), px(2, 1), px(3, 0), px(3, 1),
                  full((C, C)), full((C, C)), full((C, C)), full((C, C))],
        out_specs=[row(C), row(3 * C), row(3 * HC)],
        out_shape=[jax.ShapeDtypeStruct((NA, C), jnp.float32),
                   jax.ShapeDtypeStruct((NA, 3 * C), jnp.float32),
                   jax.ShapeDtypeStruct((NA, 3 * HC), jnp.int32)],
    )(p1, p3f, acc, acc, acc, acc, acc, acc, acc, acc,
      wp0, wp1, we, dw)


def kernel(ind_2, p1, p3, basis, d3, params):
    ind_2 = ind_2.astype(jnp.int32)
    i = ind_2[:, 0]
    j = ind_2[:, 1]
    npad = NPAD - NP
    ii_g = jnp.concatenate(
        [i, jnp.zeros((npad,), jnp.int32)]).reshape(NPAD // 64, 64)
    jj_g = jnp.concatenate(
        [j, jnp.zeros((npad,), jnp.int32)]).reshape(NPAD // 64, 64)
    ii_s = jnp.concatenate(
        [i, jnp.full((npad,), NA, jnp.int32)]).reshape(NCHUNK, 128)
    basis_p = jnp.concatenate(
        [basis, jnp.zeros((npad, NB), jnp.float32)], axis=0)
    d3_p = jnp.concatenate([
        jnp.concatenate([d3, jnp.zeros((npad, 3), jnp.float32)], axis=0),
        jnp.zeros((NPAD, NB - 3), jnp.float32)], axis=1)
    zeros = jnp.zeros((RPT, C), jnp.float32)
    p3f = p3.reshape(NA, 3 * C)
    p3fp = jnp.concatenate(
        [_pack2(p3f[:, x * C:x * C + HC], p3f[:, x * C + HC:(x + 1) * C])
         for x in range(3)], axis=1)

    for blk in params:
        h, hp = _atom_mlp(p1, blk["pp_pre_W0"], blk["pp_pre_b0"],
                          blk["pp_pre_W1"], blk["pp_pre_b1"])
        hi = _gather_hi(h, ii_g)
        tbl = jnp.concatenate([p3fp, hp], axis=1)
        jg = _gather_j(tbl, jj_g)
        i1, ix0, ix1, ix2 = _pair_net(hi, jg, basis_p, d3_p,
                                      blk["pi_W"][:C], blk["pi_W"][C:],
                                      blk["pi_b"], blk["ii_W"])
        acc = _scatter_sc(_sall_body, 4, "sc_scatter",
                          (i1, ix0, ix1, ix2), ii_s, zeros)
        p1, p3f, p3fp = _combine(p1, p3f, acc,
                                 blk["pp_post_W0"], blk["pp_post_W1"],
                                 blk["eq_pp_W"], blk["dot_W"])
    return p1


# speedup vs baseline: 1.0404x; 1.0404x over previous
import jax
import jax.numpy as jnp
from jax import lax
from jax.experimental import pallas as pl
from jax.experimental.pallas import tpu as pltpu
from jax.experimental.pallas import tpu_sc as plsc

NA = 10000
NP = 160000
C = 128
PIW = 64
NB = 8
NPAD = 163840
NCHUNK = NPAD // 128
NA_ACC = 10112
RPT = NA_ACC // 16
CPW = NCHUNK // 32
CPW64 = (NPAD // 64) // 32
NCH64 = NPAD // 64
CH_C0 = 640
TPC0 = CH_C0 // 16
TPC1 = (NCH64 - CH_C0) // 16
PAIR_BLK = 2048
ATOM_BLK = 1000
HC = C // 2

_SC_MESH = dict(core_axis_name="c", subcore_axis_name="s")


def _rn16(v):
    return (lax.bitcast_convert_type(v, jnp.int32) + 0x8000) >> 16


def _pack2(lo, hi):
    return (_rn16(hi) << 16) | (_rn16(lo) & 0xFFFF)


def _unpack2(p):
    lo = lax.bitcast_convert_type(p << 16, jnp.float32)
    hi = lax.bitcast_convert_type(p & jnp.int32(-65536), jnp.float32)
    return lo, hi


def _drain(dummy_hbm, dst_ref, sem):
    pltpu.make_async_copy(dummy_hbm, dst_ref, sem).wait()


def _gi_body(h_hbm, ii_hbm, oi_hbm, ii_v, bi, sg, sw):
    c = lax.axis_index("c")
    s = lax.axis_index("s")
    ch0 = (s * 2 + c) * CPW64
    pltpu.sync_copy(ii_hbm.at[pl.ds(ch0, CPW64)], ii_v)
    dummy = h_hbm.at[pl.ds(0, 64)]

    @pl.loop(0, CPW64, step=8)
    def _(k):
        for b in range(8):
            pltpu.async_copy(h_hbm.at[ii_v.at[k + b]], bi.at[b], sg)
        for b in range(8):
            _drain(dummy, bi.at[b], sg)
        for b in range(8):
            row = (ch0 + k + b) * 64
            pltpu.async_copy(bi.at[b], oi_hbm.at[pl.ds(row, 64)], sw)
        for b in range(8):
            _drain(dummy, bi.at[b], sw)


def _gather_hi(h, ii_g64):
    return pl.kernel(
        _gi_body,
        out_type=jax.ShapeDtypeStruct((NPAD, C), jnp.float32),
        mesh=plsc.VectorSubcoreMesh(**_SC_MESH),
        scratch_types=[
            pltpu.VMEM((CPW64, 64), jnp.int32),
            pltpu.VMEM((8, 64, C), jnp.float32),
            pltpu.SemaphoreType.DMA,
            pltpu.SemaphoreType.DMA,
        ],
        name="sc_gather_hi",
    )(h, ii_g64)


def _gj_body(t_hbm, jj_hbm, out_hbm, jj_v, buf, sg, sw):
    c = lax.axis_index("c")
    s = lax.axis_index("s")
    ch0 = (s * 2 + c) * CPW64
    pltpu.sync_copy(jj_hbm.at[pl.ds(ch0, CPW64)], jj_v)
    dummy = t_hbm.at[pl.ds(0, 64)]

    @pl.loop(0, CPW64, step=4)
    def _(k):
        for b in range(4):
            pltpu.async_copy(t_hbm.at[jj_v.at[k + b]], buf.at[b], sg)
        for b in range(4):
            _drain(dummy, buf.at[b], sg)
        for b in range(4):
            row = (ch0 + k + b) * 64
            pltpu.async_copy(buf.at[b], out_hbm.at[pl.ds(row, 64)], sw)
        for b in range(4):
            _drain(dummy, buf.at[b], sw)


def _gather_j(tbl, jj_g64):
    return pl.kernel(
        _gj_body,
        out_type=jax.ShapeDtypeStruct((NPAD, 4 * HC), jnp.int32),
        mesh=plsc.VectorSubcoreMesh(**_SC_MESH),
        scratch_types=[
            pltpu.VMEM((CPW64, 64), jnp.int32),
            pltpu.VMEM((4, 64, 4 * HC), jnp.int32),
            pltpu.SemaphoreType.DMA,
            pltpu.SemaphoreType.DMA,
        ],
        name="sc_gather_j",
    )(tbl, jj_g64)


def _scatter_rounds(vxs, ii_hbm, zeros, out, idx_v, vb, acc, sl, ss):
    c = lax.axis_index("c")
    s = lax.axis_index("s")
    r0 = s * RPT
    ch0 = (c * 16 + s) * CPW
    pltpu.sync_copy(ii_hbm.at[pl.ds(ch0, CPW)], idx_v)
    for x, vx in enumerate(vxs):
        pltpu.sync_copy(zeros, acc.at[pl.ds(r0, RPT)])
        plsc.subcore_barrier()
        dummy = vx.at[pl.ds(0, 128)]

        @pl.loop(0, CPW, step=2)
        def _(k, vx=vx, dummy=dummy):
            for b in range(2):
                pltpu.async_copy(
                    vx.at[pl.ds((ch0 + k + b) * 128, 128)], vb.at[b], sl)
            for b in range(2):
                _drain(dummy, vb.at[b], sl)
            for b in range(2):
                pltpu.async_copy(vb.at[b], acc.at[idx_v.at[k + b]], ss,
                                 add=True)
            for b in range(2):
                _drain(dummy, vb.at[b], ss)

        plsc.subcore_barrier()
        pltpu.sync_copy(acc.at[pl.ds(r0, RPT)], out.at[x, c, pl.ds(r0, RPT)])


def _sall_body(i1, x0, x1, x2, ii_hbm, zeros, out, idx_v, vb, acc, sl, ss):
    _scatter_rounds((i1, x0, x1, x2), ii_hbm, zeros, out, idx_v, vb, acc,
                    sl, ss)


def _scatter_sc(body, n, name, vals, ii_s, zeros):
    return pl.kernel(
        body,
        out_type=jax.ShapeDtypeStruct((n, 2, NA_ACC, C), jnp.float32),
        mesh=plsc.VectorSubcoreMesh(**_SC_MESH),
        scratch_types=[
            pltpu.VMEM((CPW, 128), jnp.int32),
            pltpu.VMEM((2, 128, C), jnp.float32),
            pltpu.VMEM_SHARED((NA_ACC, C), jnp.float32),
            pltpu.SemaphoreType.DMA,
            pltpu.SemaphoreType.DMA,
        ],
        name=name,
    )(*vals, ii_s, zeros)


def _t1_body(x, w0, b0, w1, b1, o, op):
    h = jnp.tanh(jnp.dot(x[...], w0[...], preferred_element_type=jnp.float32)
                 + b0[...])
    h = jnp.tanh(jnp.dot(h, w1[...], preferred_element_type=jnp.float32)
                 + b1[...])
    o[...] = h
    op[...] = _pack2(h[:, :HC], h[:, HC:])


def _atom_mlp(p1, w0, b0, w1, b1):
    full = lambda shape: pl.BlockSpec(shape, lambda i: (0, 0))
    return pl.pallas_call(
        _t1_body,
        grid=(NA // ATOM_BLK,),
        in_specs=[pl.BlockSpec((ATOM_BLK, C), lambda i: (i, 0)),
                  full((C, C)), full((1, C)), full((C, C)), full((1, C))],
        out_specs=[pl.BlockSpec((ATOM_BLK, C), lambda i: (i, 0)),
                   pl.BlockSpec((ATOM_BLK, HC), lambda i: (i, 0))],
        out_shape=[jax.ShapeDtypeStruct((NA, C), jnp.float32),
                   jax.ShapeDtypeStruct((NA, HC), jnp.int32)],
    )(p1, w0, b0.reshape(1, C), w1, b1.reshape(1, C))


def _t2_body(hi, jblk, basis, d3p, wa, wb, pib, iiw, o, o0, o1, o2):
    jv = jblk[...]
    hjl, hjh = _unpack2(jv[:, 3 * HC:])
    hj = jnp.concatenate([hjl, hjh], axis=1).astype(jnp.bfloat16)
    hib = hi[...].astype(jnp.bfloat16)
    pre = (jnp.dot(hib, wa[...], preferred_element_type=jnp.float32)
           + jnp.dot(hj, wb[...], preferred_element_type=jnp.float32))
    t = jnp.tanh(pre + pib[...])
    r8 = lax.broadcasted_iota(jnp.int32, (NB, PIW * NB), 0)
    c8 = lax.broadcasted_iota(jnp.int32, (NB, PIW * NB), 1)
    em = (c8 % NB == r8).astype(jnp.float32)
    rw = lax.broadcasted_iota(jnp.int32, (PIW * NB, PIW), 1)
    cw = lax.broadcasted_iota(jnp.int32, (PIW * NB, PIW), 0)
    fm = (cw // NB == rw).astype(jnp.bfloat16)
    y = (t * jnp.dot(basis[...], em, preferred_element_type=jnp.float32)
         ).astype(jnp.bfloat16)
    ip = jnp.dot(y, fm, preferred_element_type=jnp.float32)
    i1v = jnp.tanh(jnp.dot(ip, iiw[...], preferred_element_type=jnp.float32))
    o[...] = i1v
    d3v = d3p[...]
    for x, ox in enumerate((o0, o1, o2)):
        lo, hi_ = _unpack2(jv[:, x * HC:(x + 1) * HC])
        comp = jnp.concatenate([lo, hi_], axis=1)
        ox[...] = (comp + d3v[:, x:x + 1]) * i1v


def _pair_net(hi, jg, basis_p, d3_p, wa, wb, pib, iiw):
    full = lambda shape: pl.BlockSpec(shape, lambda i: (0, 0))
    row = lambda w: pl.BlockSpec((PAIR_BLK, w), lambda i: (i, 0))
    return pl.pallas_call(
        _t2_body,
        grid=(NPAD // PAIR_BLK,),
        in_specs=[row(C), row(4 * HC), row(NB), row(NB),
                  full((C, PIW * NB)), full((C, PIW * NB)),
                  full((1, PIW * NB)), full((PIW, C))],
        out_specs=[row(C), row(C), row(C), row(C)],
        out_shape=[jax.ShapeDtypeStruct((NPAD, C), jnp.float32)] * 4,
    )(hi, jg, basis_p, d3_p, wa.astype(jnp.bfloat16),
      wb.astype(jnp.bfloat16), pib.reshape(1, PIW * NB), iiw)


def _t3_body(p1, p3f, pa0, pa1, px00, px01, px10, px11, px20, px21,
             wp0, wp1, we, dw, o1, o3, o3p):
    p1a = pa0[0, 0] + pa1[0, 0]
    p1n = jnp.tanh(jnp.dot(p1a, wp0[...], preferred_element_type=jnp.float32))
    p1n = jnp.tanh(jnp.dot(p1n, wp1[...], preferred_element_type=jnp.float32))
    wev = we[...]
    p3n = [jnp.dot(a[0, 0] + b[0, 0], wev, preferred_element_type=jnp.float32)
           for a, b in ((px00, px01), (px10, px11), (px20, px21))]
    dotted = p3n[0] * p3n[0] + p3n[1] * p3n[1] + p3n[2] * p3n[2]
    p1o = p1n + jnp.dot(dotted, dw[...], preferred_element_type=jnp.float32)
    o1[...] = p1[...] + p1o
    p3fv = p3f[...]
    news = [p3fv[:, x * C:(x + 1) * C] + p3n[x] for x in range(3)]
    o3[...] = jnp.concatenate(news, axis=1)
    o3p[...] = jnp.concatenate(
        [_pack2(n[:, :HC], n[:, HC:]) for n in news], axis=1)


def _combine(p1, p3f, acc, wp0, wp1, we, dw):
    full = lambda shape: pl.BlockSpec(shape, lambda i: (0, 0))
    row = lambda w: pl.BlockSpec((ATOM_BLK, w), lambda i: (i, 0))
    px = lambda xx, cc: pl.BlockSpec(
        (1, 1, ATOM_BLK, C), lambda i, xx=xx, cc=cc: (xx, cc, i, 0))
    return pl.pallas_call(
        _t3_body,
        grid=(NA // ATOM_BLK,),
        in_specs=[row(C), row(3 * C), px(0, 0), px(0, 1),
                  px(1, 0), px(1, 1), px(2, 0), px(2, 1), px(3, 0), px(3, 1),
                  full((C, C)), full((C, C)), full((C, C)), full((C, C))],
        out_specs=[row(C), row(3 * C), row(3 * HC)],
        out_shape=[jax.ShapeDtypeStruct((NA, C), jnp.float32),
                   jax.ShapeDtypeStruct((NA, 3 * C), jnp.float32),
                   jax.ShapeDtypeStruct((NA, 3 * HC), jnp.int32)],
    )(p1, p3f, acc, acc, acc, acc, acc, acc, acc, acc,
      wp0, wp1, we, dw)


def kernel(ind_2, p1, p3, basis, d3, params):
    ind_2 = ind_2.astype(jnp.int32)
    i = ind_2[:, 0]
    j = ind_2[:, 1]
    npad = NPAD - NP
    ii_g = jnp.concatenate(
        [i, jnp.zeros((npad,), jnp.int32)]).reshape(NPAD // 64, 64)
    jj_g = jnp.concatenate(
        [j, jnp.zeros((npad,), jnp.int32)]).reshape(NPAD // 64, 64)
    ii_s = jnp.concatenate(
        [i, jnp.full((npad,), NA, jnp.int32)]).reshape(NCHUNK, 128)
    basis_p = jnp.concatenate(
        [basis, jnp.zeros((npad, NB), jnp.float32)], axis=0)
    d3_p = jnp.concatenate([
        jnp.concatenate([d3, jnp.zeros((npad, 3), jnp.float32)], axis=0),
        jnp.zeros((NPAD, NB - 3), jnp.float32)], axis=1)
    zeros = jnp.zeros((RPT, C), jnp.float32)
    p3f = p3.reshape(NA, 3 * C)
    p3fp = jnp.concatenate(
        [_pack2(p3f[:, x * C:x * C + HC], p3f[:, x * C + HC:(x + 1) * C])
         for x in range(3)], axis=1)

    for blk in params:
        h, hp = _atom_mlp(p1, blk["pp_pre_W0"], blk["pp_pre_b0"],
                          blk["pp_pre_W1"], blk["pp_pre_b1"])
        hi = _gather_hi(h, ii_g)
        tbl = jnp.concatenate([p3fp, hp], axis=1)
        jg = _gather_j(tbl, jj_g)
        i1, ix0, ix1, ix2 = _pair_net(hi, jg, basis_p, d3_p,
                                      blk["pi_W"][:C], blk["pi_W"][C:],
                                      blk["pi_b"], blk["ii_W"])
        acc = _scatter_sc(_sall_body, 4, "sc_scatter",
                          (i1, ix0, ix1, ix2), ii_s, zeros)
        p1, p3f, p3fp = _combine(p1, p3f, acc,
                                 blk["pp_post_W0"], blk["pp_post_W1"],
                                 blk["eq_pp_W"], blk["dot_W"])
    return p1


# speedup vs baseline: 1.1288x; 1.0850x over previous
import jax
import jax.numpy as jnp
from jax import lax
from jax.experimental import pallas as pl
from jax.experimental.pallas import tpu as pltpu
from jax.experimental.pallas import tpu_sc as plsc

NA = 10000
NP = 160000
C = 128
PIW = 64
NB = 8
NPAD = 163840
NCHUNK = NPAD // 128
NA_ACC = 10112
RPT = NA_ACC // 16
CPW = NCHUNK // 32
NPH = NPAD // 2
CPW64H = (NPH // 64) // 32
NCH_H = NPH // 128
CPTH = NCH_H // 32
PAIR_BLK = 2048
ATOM_BLK = 1000
HC = C // 2

_SC_MESH = dict(core_axis_name="c", subcore_axis_name="s")


def _rn16(v):
    return (lax.bitcast_convert_type(v, jnp.int32) + 0x8000) >> 16


def _pack2(lo, hi):
    return (_rn16(hi) << 16) | (_rn16(lo) & 0xFFFF)


def _unpack2(p):
    lo = lax.bitcast_convert_type(p << 16, jnp.float32)
    hi = lax.bitcast_convert_type(p & jnp.int32(-65536), jnp.float32)
    return lo, hi


def _drain(dummy_hbm, dst_ref, sem):
    pltpu.make_async_copy(dummy_hbm, dst_ref, sem).wait()


def _gi_body(h_hbm, ii_hbm, oi_hbm, ii_v, bi, sg, sw):
    c = lax.axis_index("c")
    s = lax.axis_index("s")
    ch0 = (s * 2 + c) * CPW64H
    pltpu.sync_copy(ii_hbm.at[pl.ds(ch0, CPW64H)], ii_v)
    dummy = h_hbm.at[pl.ds(0, 64)]

    @pl.loop(0, CPW64H, step=8)
    def _(k):
        for b in range(8):
            pltpu.async_copy(h_hbm.at[ii_v.at[k + b]], bi.at[b], sg)
        for b in range(8):
            _drain(dummy, bi.at[b], sg)
        for b in range(8):
            row = (ch0 + k + b) * 64
            pltpu.async_copy(bi.at[b], oi_hbm.at[pl.ds(row, 64)], sw)
        for b in range(8):
            _drain(dummy, bi.at[b], sw)


def _gather_hi(h, ii_g64):
    return pl.kernel(
        _gi_body,
        out_type=jax.ShapeDtypeStruct((NPH, C), jnp.float32),
        mesh=plsc.VectorSubcoreMesh(**_SC_MESH),
        scratch_types=[
            pltpu.VMEM((CPW64H, 64), jnp.int32),
            pltpu.VMEM((8, 64, C), jnp.float32),
            pltpu.SemaphoreType.DMA,
            pltpu.SemaphoreType.DMA,
        ],
        name="sc_gather_hi",
    )(h, ii_g64)


def _gj_body(t_hbm, jj_hbm, out_hbm, jj_v, buf, sg, sw):
    c = lax.axis_index("c")
    s = lax.axis_index("s")
    ch0 = (s * 2 + c) * CPW64H
    pltpu.sync_copy(jj_hbm.at[pl.ds(ch0, CPW64H)], jj_v)
    dummy = t_hbm.at[pl.ds(0, 64)]

    @pl.loop(0, CPW64H, step=4)
    def _(k):
        for b in range(4):
            pltpu.async_copy(t_hbm.at[jj_v.at[k + b]], buf.at[b], sg)
        for b in range(4):
            _drain(dummy, buf.at[b], sg)
        for b in range(4):
            row = (ch0 + k + b) * 64
            pltpu.async_copy(buf.at[b], out_hbm.at[pl.ds(row, 64)], sw)
        for b in range(4):
            _drain(dummy, buf.at[b], sw)


def _gather_j(tbl, jj_g64):
    return pl.kernel(
        _gj_body,
        out_type=jax.ShapeDtypeStruct((NPH, 4 * HC), jnp.int32),
        mesh=plsc.VectorSubcoreMesh(**_SC_MESH),
        scratch_types=[
            pltpu.VMEM((CPW64H, 64), jnp.int32),
            pltpu.VMEM((4, 64, 4 * HC), jnp.int32),
            pltpu.SemaphoreType.DMA,
            pltpu.SemaphoreType.DMA,
        ],
        name="sc_gather_j",
    )(tbl, jj_g64)


def _scatter_rounds(vxs, ii_hbm, zeros, out, idx_v, vb, acc, sl, ss):
    c = lax.axis_index("c")
    s = lax.axis_index("s")
    r0 = s * RPT
    ch0 = (c * 16 + s) * CPTH
    start = (ch0 // 8) * 8
    off = ch0 - start
    pltpu.sync_copy(ii_hbm.at[pl.ds(start, 32)], idx_v)
    for x, vx in enumerate(vxs):
        pltpu.sync_copy(zeros, acc.at[pl.ds(r0, RPT)])
        plsc.subcore_barrier()
        dummy = vx.at[pl.ds(0, 128)]

        @pl.loop(0, CPTH, step=2)
        def _(k, vx=vx, dummy=dummy):
            for b in range(2):
                pltpu.async_copy(
                    vx.at[pl.ds((ch0 + k + b) * 128, 128)], vb.at[b], sl)
            for b in range(2):
                _drain(dummy, vb.at[b], sl)
            for b in range(2):
                pltpu.async_copy(vb.at[b], acc.at[idx_v.at[off + k + b]], ss,
                                 add=True)
            for b in range(2):
                _drain(dummy, vb.at[b], ss)

        plsc.subcore_barrier()
        pltpu.sync_copy(acc.at[pl.ds(r0, RPT)], out.at[x, c, pl.ds(r0, RPT)])


def _sall_body(i1, x0, x1, x2, ii_hbm, zeros, out, idx_v, vb, acc, sl, ss):
    _scatter_rounds((i1, x0, x1, x2), ii_hbm, zeros, out, idx_v, vb, acc,
                    sl, ss)


def _scatter_sc(body, n, name, vals, ii_s, zeros):
    return pl.kernel(
        body,
        out_type=jax.ShapeDtypeStruct((n, 2, NA_ACC, C), jnp.float32),
        mesh=plsc.VectorSubcoreMesh(**_SC_MESH),
        scratch_types=[
            pltpu.VMEM((32, 128), jnp.int32),
            pltpu.VMEM((2, 128, C), jnp.float32),
            pltpu.VMEM_SHARED((NA_ACC, C), jnp.float32),
            pltpu.SemaphoreType.DMA,
            pltpu.SemaphoreType.DMA,
        ],
        name=name,
    )(*vals, ii_s, zeros)


def _t1_body(x, w0, b0, w1, b1, o, op):
    h = jnp.tanh(jnp.dot(x[...], w0[...], preferred_element_type=jnp.float32)
                 + b0[...])
    h = jnp.tanh(jnp.dot(h, w1[...], preferred_element_type=jnp.float32)
                 + b1[...])
    o[...] = h
    op[...] = _pack2(h[:, :HC], h[:, HC:])


def _atom_mlp(p1, w0, b0, w1, b1):
    full = lambda shape: pl.BlockSpec(shape, lambda i: (0, 0))
    return pl.pallas_call(
        _t1_body,
        grid=(NA // ATOM_BLK,),
        in_specs=[pl.BlockSpec((ATOM_BLK, C), lambda i: (i, 0)),
                  full((C, C)), full((1, C)), full((C, C)), full((1, C))],
        out_specs=[pl.BlockSpec((ATOM_BLK, C), lambda i: (i, 0)),
                   pl.BlockSpec((ATOM_BLK, HC), lambda i: (i, 0))],
        out_shape=[jax.ShapeDtypeStruct((NA, C), jnp.float32),
                   jax.ShapeDtypeStruct((NA, HC), jnp.int32)],
    )(p1, w0, b0.reshape(1, C), w1, b1.reshape(1, C))


def _t2_body(hi, jblk, basis, d3p, wa, wb, pib, iiw, o, o0, o1, o2):
    jv = jblk[...]
    hjl, hjh = _unpack2(jv[:, 3 * HC:])
    hj = jnp.concatenate([hjl, hjh], axis=1).astype(jnp.bfloat16)
    hib = hi[...].astype(jnp.bfloat16)
    pre = (jnp.dot(hib, wa[...], preferred_element_type=jnp.float32)
           + jnp.dot(hj, wb[...], preferred_element_type=jnp.float32))
    t = jnp.tanh(pre + pib[...])
    r8 = lax.broadcasted_iota(jnp.int32, (NB, PIW * NB), 0)
    c8 = lax.broadcasted_iota(jnp.int32, (NB, PIW * NB), 1)
    em = (c8 % NB == r8).astype(jnp.float32)
    rw = lax.broadcasted_iota(jnp.int32, (PIW * NB, PIW), 1)
    cw = lax.broadcasted_iota(jnp.int32, (PIW * NB, PIW), 0)
    fm = (cw // NB == rw).astype(jnp.bfloat16)
    y = (t * jnp.dot(basis[...], em, preferred_element_type=jnp.float32)
         ).astype(jnp.bfloat16)
    ip = jnp.dot(y, fm, preferred_element_type=jnp.float32)
    i1v = jnp.tanh(jnp.dot(ip, iiw[...], preferred_element_type=jnp.float32))
    o[...] = i1v
    d3v = d3p[...]
    for x, ox in enumerate((o0, o1, o2)):
        lo, hi_ = _unpack2(jv[:, x * HC:(x + 1) * HC])
        comp = jnp.concatenate([lo, hi_], axis=1)
        ox[...] = (comp + d3v[:, x:x + 1]) * i1v


def _pair_net(hi, jg, basis_p, d3_p, wa, wb, pib, iiw):
    full = lambda shape: pl.BlockSpec(shape, lambda i: (0, 0))
    row = lambda w: pl.BlockSpec((PAIR_BLK, w), lambda i: (i, 0))
    return pl.pallas_call(
        _t2_body,
        grid=(NPH // PAIR_BLK,),
        in_specs=[row(C), row(4 * HC), row(NB), row(NB),
                  full((C, PIW * NB)), full((C, PIW * NB)),
                  full((1, PIW * NB)), full((PIW, C))],
        out_specs=[row(C), row(C), row(C), row(C)],
        out_shape=[jax.ShapeDtypeStruct((NPH, C), jnp.float32)] * 4,
    )(hi, jg, basis_p, d3_p, wa.astype(jnp.bfloat16),
      wb.astype(jnp.bfloat16), pib.reshape(1, PIW * NB), iiw)


def _t3_body(p1, p3f, pa0, pa1, pb0, pb1,
             px00, px01, px10, px11, px20, px21,
             py00, py01, py10, py11, py20, py21,
             wp0, wp1, we, dw, o1, o3, o3p):
    p1a = pa0[0, 0] + pa1[0, 0] + pb0[0, 0] + pb1[0, 0]
    p1n = jnp.tanh(jnp.dot(p1a, wp0[...], preferred_element_type=jnp.float32))
    p1n = jnp.tanh(jnp.dot(p1n, wp1[...], preferred_element_type=jnp.float32))
    wev = we[...]
    p3n = [jnp.dot(a[0, 0] + b[0, 0] + aa[0, 0] + bb[0, 0], wev,
                   preferred_element_type=jnp.float32)
           for a, b, aa, bb in ((px00, px01, py00, py01),
                                (px10, px11, py10, py11),
                                (px20, px21, py20, py21))]
    dotted = p3n[0] * p3n[0] + p3n[1] * p3n[1] + p3n[2] * p3n[2]
    p1o = p1n + jnp.dot(dotted, dw[...], preferred_element_type=jnp.float32)
    o1[...] = p1[...] + p1o
    p3fv = p3f[...]
    news = [p3fv[:, x * C:(x + 1) * C] + p3n[x] for x in range(3)]
    o3[...] = jnp.concatenate(news, axis=1)
    o3p[...] = jnp.concatenate(
        [_pack2(n[:, :HC], n[:, HC:]) for n in news], axis=1)


def _combine(p1, p3f, accA, accB, wp0, wp1, we, dw):
    full = lambda shape: pl.BlockSpec(shape, lambda i: (0, 0))
    row = lambda w: pl.BlockSpec((ATOM_BLK, w), lambda i: (i, 0))
    px = lambda xx, cc: pl.BlockSpec(
        (1, 1, ATOM_BLK, C), lambda i, xx=xx, cc=cc: (xx, cc, i, 0))
    return pl.pallas_call(
        _t3_body,
        grid=(NA // ATOM_BLK,),
        in_specs=[row(C), row(3 * C),
                  px(0, 0), px(0, 1), px(0, 0), px(0, 1),
                  px(1, 0), px(1, 1), px(2, 0), px(2, 1), px(3, 0), px(3, 1),
                  px(1, 0), px(1, 1), px(2, 0), px(2, 1), px(3, 0), px(3, 1),
                  full((C, C)), full((C, C)), full((C, C)), full((C, C))],
        out_specs=[row(C), row(3 * C), row(3 * HC)],
        out_shape=[jax.ShapeDtypeStruct((NA, C), jnp.float32),
                   jax.ShapeDtypeStruct((NA, 3 * C), jnp.float32),
                   jax.ShapeDtypeStruct((NA, 3 * HC), jnp.int32)],
    )(p1, p3f, accA, accA, accB, accB,
      accA, accA, accA, accA, accA, accA,
      accB, accB, accB, accB, accB, accB,
      wp0, wp1, we, dw)


def kernel(ind_2, p1, p3, basis, d3, params):
    ind_2 = ind_2.astype(jnp.int32)
    i = ind_2[:, 0]
    j = ind_2[:, 1]
    npad = NPAD - NP
    ii_g = jnp.concatenate(
        [i, jnp.zeros((npad,), jnp.int32)]).reshape(NPAD // 64, 64)
    jj_g = jnp.concatenate(
        [j, jnp.zeros((npad,), jnp.int32)]).reshape(NPAD // 64, 64)
    ii_s = jnp.concatenate(
        [i, jnp.full((npad,), NA, jnp.int32)]).reshape(NCHUNK, 128)
    basis_p = jnp.concatenate(
        [basis, jnp.zeros((npad, NB), jnp.float32)], axis=0)
    d3_p = jnp.concatenate([
        jnp.concatenate([d3, jnp.zeros((npad, 3), jnp.float32)], axis=0),
        jnp.zeros((NPAD, NB - 3), jnp.float32)], axis=1)
    zeros = jnp.zeros((RPT, C), jnp.float32)
    p3f = p3.reshape(NA, 3 * C)
    p3fp = jnp.concatenate(
        [_pack2(p3f[:, x * C:x * C + HC], p3f[:, x * C + HC:(x + 1) * C])
         for x in range(3)], axis=1)

    iig_h = [ii_g[:NPH // 64], ii_g[NPH // 64:]]
    jjg_h = [jj_g[:NPH // 64], jj_g[NPH // 64:]]
    pad8 = jnp.full((8, 128), NA, jnp.int32)
    iis_h = [jnp.concatenate([ii_s[:NCH_H], pad8]),
             jnp.concatenate([ii_s[NCH_H:], pad8])]
    basis_h = [basis_p[:NPH], basis_p[NPH:]]
    d3_h = [d3_p[:NPH], d3_p[NPH:]]

    for blk in params:
        h, hp = _atom_mlp(p1, blk["pp_pre_W0"], blk["pp_pre_b0"],
                          blk["pp_pre_W1"], blk["pp_pre_b1"])
        tbl = jnp.concatenate([p3fp, hp], axis=1)
        gath = []
        for hx in range(2):
            hi = _gather_hi(h, iig_h[hx])
            jg = _gather_j(tbl, jjg_h[hx])
            gath.append((hi, jg))
        accs = []
        for hx in range(2):
            hi, jg = gath[hx]
            i1, ix0, ix1, ix2 = _pair_net(hi, jg, basis_h[hx], d3_h[hx],
                                          blk["pi_W"][:C], blk["pi_W"][C:],
                                          blk["pi_b"], blk["ii_W"])
            accs.append(_scatter_sc(_sall_body, 4, "sc_scatter",
                                    (i1, ix0, ix1, ix2), iis_h[hx], zeros))
        p1, p3f, p3fp = _combine(p1, p3f, accs[0], accs[1],
                                 blk["pp_post_W0"], blk["pp_post_W1"],
                                 blk["eq_pp_W"], blk["dot_W"])
    return p1
